# baseline (speedup 1.0000x reference)
import functools

import jax
import jax.numpy as jnp
from jax import lax
from jax.experimental import pallas as pl
from jax.experimental.pallas import tpu as pltpu
from jax.experimental.pallas import tpu_sc as plsc

BATCH = 16384
EMBED = 256
LANES = 16
COLV = EMBED // LANES
NC = 2
NS = 16
NW = NC * NS

SC_ROWS = 6144
TC_ROWS = BATCH - SC_ROWS
BPW = SC_ROWS // NW
NCHUNK = 2
NBUF = 2
CH = BPW // NCHUNK
GRP = CH // LANES

BLK = 2048

_mesh = plsc.VectorSubcoreMesh(core_axis_name="c", subcore_axis_name="s")


@functools.partial(
    pl.kernel,
    mesh=_mesh,
    out_type=jax.ShapeDtypeStruct((BATCH, EMBED), jnp.float32),
    scratch_types=[
        pltpu.VMEM((NCHUNK, CH), jnp.int32),
        pltpu.VMEM((2, EMBED), jnp.float32),
        pltpu.VMEM((CH, EMBED), jnp.float32),
        pltpu.VMEM((CH, EMBED), jnp.float32),
        pltpu.SemaphoreType.DMA,
        pltpu.SemaphoreType.DMA,
    ],
)
def _embed_fill(idx_hbm, table_hbm, out_hbm, idx_v, tab_v,
                rows0, rows1, ssem0, ssem1):
    wid = lax.axis_index("s") * NC + lax.axis_index("c")
    base = wid * BPW

    pltpu.sync_copy(idx_hbm.at[wid], idx_v)
    pltpu.sync_copy(table_hbm, tab_v)

    for j in range(COLV):
        s = pl.ds(LANES * j, LANES)
        tab_v[1, s] = tab_v[1, s] - tab_v[0, s]

    _dn = lax.GatherDimensionNumbers(
        offset_dims=(), collapsed_slice_dims=(0,), start_index_map=(0,))

    def lane_bcast(x, r):
        idx = jnp.full((LANES, 1), r, jnp.int32)
        return lax.gather(x, idx, _dn, slice_sizes=(1,),
                          mode=lax.GatherScatterMode.PROMISE_IN_BOUNDS)

    bufs = (rows0, rows1)
    ssems = (ssem0, ssem1)
    stores = [None] * NBUF

    for c in range(NCHUNK):
        p = c % NBUF
        if stores[p] is not None:
            stores[p].wait()
            stores[p] = None
        buf = bufs[p]

        def fill_group(g, _, c=c, buf=buf):
            fv = idx_v[c, pl.ds(g * LANES, LANES)].astype(jnp.float32)
            fs = [lane_bcast(fv, r) for r in range(LANES)]
            rowbase = g * LANES
            for j in range(COLV):
                s = pl.ds(LANES * j, LANES)
                a = tab_v[0, s]
                d = tab_v[1, s]
                for r in range(LANES):
                    buf[rowbase + r, s] = a + fs[r] * d
            return 0

        lax.fori_loop(0, GRP, fill_group, 0)
        stores[p] = pltpu.async_copy(
            buf, out_hbm.at[pl.ds(base + c * CH, CH)], ssems[p])

    for s in stores:
        if s is not None:
            s.wait()


def _tc_body(acc_ref, idx_ref, tab_ref, o_ref):
    del acc_ref
    f = idx_ref[...].astype(jnp.float32)
    r0 = tab_ref[0:1, :]
    d = tab_ref[1:2, :] - tab_ref[0:1, :]
    o_ref[...] = r0 + f * d


def _tc_fill(acc, idx2d, table):
    return pl.pallas_call(
        _tc_body,
        grid=(TC_ROWS // BLK,),
        in_specs=[
            pl.BlockSpec(memory_space=pl.ANY),
            pl.BlockSpec((BLK, 1), lambda i: (i, 0)),
            pl.BlockSpec((2, EMBED), lambda i: (0, 0)),
        ],
        out_specs=pl.BlockSpec((BLK, EMBED),
                               lambda i: (i + SC_ROWS // BLK, 0)),
        out_shape=jax.ShapeDtypeStruct((BATCH, EMBED), jnp.float32),
        input_output_aliases={0: 0},
    )(acc, idx2d, table)


def kernel(input, table):
    idx = jnp.asarray(input, jnp.int32)
    acc = _embed_fill(idx[:SC_ROWS].reshape(NW, NCHUNK, CH), table)
    return _tc_fill(acc, idx[SC_ROWS:].reshape(TC_ROWS, 1), table)


# speedup vs baseline: 1.0759x; 1.0759x over previous
import functools

import jax
import jax.numpy as jnp
from jax import lax
from jax.experimental import pallas as pl
from jax.experimental.pallas import tpu as pltpu
from jax.experimental.pallas import tpu_sc as plsc

BATCH = 16384
EMBED = 256
LANES = 16
COLV = EMBED // LANES
NC = 2
NS = 16
NW = NC * NS
BPW = BATCH // NW
NCHUNK = 4
NBUF = 2
CH = BPW // NCHUNK
GRP = CH // LANES

_mesh = plsc.VectorSubcoreMesh(core_axis_name="c", subcore_axis_name="s")


@functools.partial(
    pl.kernel,
    mesh=_mesh,
    out_type=jax.ShapeDtypeStruct((BATCH, EMBED), jnp.float32),
    scratch_types=[
        pltpu.VMEM((NCHUNK, CH), jnp.int32),
        pltpu.VMEM((2, EMBED), jnp.float32),
        pltpu.VMEM((CH, EMBED), jnp.float32),
        pltpu.VMEM((CH, EMBED), jnp.float32),
        pltpu.SemaphoreType.DMA,
        pltpu.SemaphoreType.DMA,
    ],
)
def _embed_fill(idx_hbm, table_hbm, out_hbm, idx_v, tab_v,
                rows0, rows1, ssem0, ssem1):
    wid = lax.axis_index("s") * NC + lax.axis_index("c")
    base = wid * BPW

    pltpu.sync_copy(idx_hbm.at[wid], idx_v)
    pltpu.sync_copy(table_hbm, tab_v)

    for j in range(COLV):
        s = pl.ds(LANES * j, LANES)
        tab_v[1, s] = tab_v[1, s] - tab_v[0, s]

    _dn = lax.GatherDimensionNumbers(
        offset_dims=(), collapsed_slice_dims=(0,), start_index_map=(0,))

    def lane_bcast(x, r):
        idx = jnp.full((LANES, 1), r, jnp.int32)
        return lax.gather(x, idx, _dn, slice_sizes=(1,),
                          mode=lax.GatherScatterMode.PROMISE_IN_BOUNDS)

    bufs = (rows0, rows1)
    ssems = (ssem0, ssem1)
    stores = [None] * NBUF

    for c in range(NCHUNK):
        p = c % NBUF
        if stores[p] is not None:
            stores[p].wait()
            stores[p] = None
        buf = bufs[p]

        def fill_group(g, _, c=c, buf=buf):
            fv = idx_v[c, pl.ds(g * LANES, LANES)].astype(jnp.float32)
            fs = [lane_bcast(fv, r) for r in range(LANES)]
            rowbase = g * LANES
            for j in range(COLV):
                s = pl.ds(LANES * j, LANES)
                a = tab_v[0, s]
                d = tab_v[1, s]
                for r in range(LANES):
                    buf[rowbase + r, s] = a + fs[r] * d
            return 0

        lax.fori_loop(0, GRP, fill_group, 0)
        stores[p] = pltpu.async_copy(
            buf, out_hbm.at[pl.ds(base + c * CH, CH)], ssems[p])

    for s in stores:
        if s is not None:
            s.wait()


def kernel(input, table):
    idx = jnp.asarray(input, jnp.int32).reshape(NW, NCHUNK, CH)
    return _embed_fill(idx, table)
